# baseline (speedup 1.0000x reference)
import functools

import jax
import jax.numpy as jnp
from jax import lax
from jax.experimental import pallas as pl
from jax.experimental.pallas import tpu as pltpu
from jax.experimental.pallas import tpu_sc as plsc

SEQ_LEN = 8192
NUM_ROWS = 4
MASK_VALUE = 103.0

NUM_CORES = 2
NUM_SUBCORES = 16
LANES = 16
NUM_WORKERS = NUM_CORES * NUM_SUBCORES
COLS = SEQ_LEN // NUM_WORKERS

_mesh = plsc.VectorSubcoreMesh(core_axis_name="c", subcore_axis_name="s")


@functools.partial(
    pl.kernel,
    out_type=jax.ShapeDtypeStruct((NUM_ROWS, SEQ_LEN), jnp.float32),
    mesh=_mesh,
    scratch_types=[
        pltpu.VMEM((NUM_ROWS, COLS), jnp.float32),
        pltpu.VMEM((COLS,), jnp.int32),
    ],
)
def _mask_kernel(seqs_hbm, mask_hbm, out_hbm, seq_v, mask_v):
    wid = lax.axis_index("s") * NUM_CORES + lax.axis_index("c")
    base = wid * COLS
    pltpu.sync_copy(mask_hbm.at[pl.ds(base, COLS)], mask_v)
    pltpu.sync_copy(seqs_hbm.at[:, pl.ds(base, COLS)], seq_v)
    for i in range(COLS // LANES):
        sl = pl.ds(i * LANES, LANES)
        m = mask_v[sl] != 0
        for r in range(NUM_ROWS):
            seq_v[r, sl] = jnp.where(m, jnp.float32(MASK_VALUE), seq_v[r, sl])
    pltpu.sync_copy(seq_v, out_hbm.at[:, pl.ds(base, COLS)])


def kernel(indexed_seqs, src_mask, attn_mask):
    masked = _mask_kernel(indexed_seqs, src_mask.astype(jnp.int32))
    return (masked, attn_mask)
